# Initial kernel scaffold; baseline (speedup 1.0000x reference)
import jax
import jax.numpy as jnp
from jax.experimental import pallas as pl


def kernel(x_individuals, x_occupation, x_residence, edge_index_family, edge_index_occupation, edge_index_residence, population, edge_attributes, Wl_dir_occ, bl_dir_occ, Wr_dir_occ, Wl_dir_res, bl_dir_res, Wr_dir_res, Wl_msg, bl_msg, Wr_msg, Wl_inv_occ, bl_inv_occ, Wr_inv_occ, Wl_inv_res, bl_inv_res, Wr_inv_res, Wl_inv_ind, bl_inv_ind, Wr_inv_ind, P_occ, P_res, W_aggr, b_aggr):
    raise NotImplementedError("write your pallas kernel here")


# speedup vs baseline: 1.5807x; 1.5807x over previous
import jax
import jax.numpy as jnp
from jax import lax
from jax.experimental import pallas as pl
from jax.experimental.pallas import tpu as pltpu
from jax.experimental.pallas import tpu_sc as plsc

f32 = jnp.float32
i32 = jnp.int32

D = 128
N_IND_P = 10240
N_ATT_P = 1024
EC = 32
FAM_CH = 640
ATT_CH = 24


def _sc_agg_body(table, fsrc, fdst, osrc, odst, rsrc, rdst, zrows, ones8,
                 out_fam, out_occ, out_res,
                 acc_fam, acc_occ, acc_res,
                 isrc_g, idst_g, icur_s, icur_d, vbuf, ones_v, sem):
    c = lax.axis_index("c")
    s = lax.axis_index("s")

    pltpu.sync_copy(zrows, acc_fam.at[pl.ds(s * 640, 640)])
    pltpu.sync_copy(zrows.at[pl.ds(0, 64)], acc_occ.at[pl.ds(s * 64, 64)])
    pltpu.sync_copy(zrows.at[pl.ds(0, 64)], acc_res.at[pl.ds(s * 64, 64)])
    pltpu.sync_copy(ones8, ones_v)
    plsc.subcore_barrier()

    def run_edges(src_hbm, dst_hbm, acc, nchunk, gsz):
        def outer(g, carry):
            pltpu.sync_copy(src_hbm.at[s, pl.ds(g * gsz, gsz)],
                            isrc_g.at[pl.ds(0, gsz)])
            pltpu.sync_copy(dst_hbm.at[s, pl.ds(g * gsz, gsz)],
                            idst_g.at[pl.ds(0, gsz)])

            def inner(jj, carry2):
                for l in range(EC // 16):
                    icur_s[pl.ds(l * 16, 16)] = isrc_g[jj, pl.ds(l * 16, 16)]
                    icur_d[pl.ds(l * 16, 16)] = idst_g[jj, pl.ds(l * 16, 16)]

                @pl.when(c == 0)
                def _():
                    pltpu.async_copy(table.at[icur_s], vbuf, sem).wait()
                    pltpu.sync_copy(vbuf, acc.at[icur_d], add=True)

                @pl.when(c == 1)
                def _():
                    pltpu.sync_copy(ones_v, acc.at[icur_d], add=True)

                return carry2

            lax.fori_loop(0, gsz, inner, 0)
            return carry

        lax.fori_loop(0, nchunk // gsz, outer, 0)

    run_edges(fsrc, fdst, acc_fam, FAM_CH, 4)
    run_edges(osrc, odst, acc_occ, ATT_CH, 4)
    run_edges(rsrc, rdst, acc_res, ATT_CH, 4)
    plsc.subcore_barrier()

    pltpu.sync_copy(acc_fam.at[pl.ds(s * 640, 640)],
                    out_fam.at[c, pl.ds(s * 640, 640)])
    pltpu.sync_copy(acc_occ.at[pl.ds(s * 64, 64)],
                    out_occ.at[c, pl.ds(s * 64, 64)])
    pltpu.sync_copy(acc_res.at[pl.ds(s * 64, 64)],
                    out_res.at[c, pl.ds(s * 64, 64)])


_sc_agg = pl.kernel(
    _sc_agg_body,
    out_type=(
        jax.ShapeDtypeStruct((2, N_IND_P, D), f32),
        jax.ShapeDtypeStruct((2, N_ATT_P, D), f32),
        jax.ShapeDtypeStruct((2, N_ATT_P, D), f32),
    ),
    mesh=plsc.VectorSubcoreMesh(core_axis_name="c", subcore_axis_name="s",
                                num_cores=2, num_subcores=16),
    scratch_types=[
        pltpu.VMEM_SHARED((N_IND_P, D), f32),
        pltpu.VMEM_SHARED((N_ATT_P, D), f32),
        pltpu.VMEM_SHARED((N_ATT_P, D), f32),
        pltpu.VMEM((4, EC), i32),
        pltpu.VMEM((4, EC), i32),
        pltpu.VMEM((EC,), i32),
        pltpu.VMEM((EC,), i32),
        pltpu.VMEM((EC, D), f32),
        pltpu.VMEM((EC, D), f32),
        pltpu.SemaphoreType.DMA,
    ],
)


def _mean(ref):
    return ref[0] * (1.0 / jnp.maximum(ref[1][:, 0:1], 1.0))


def _dot(a, b):
    return jax.lax.dot(a, b, preferred_element_type=f32)


def _tc1_body(fam, occ, res, xi, xo, xr,
              Wlm, blm, Wrm, Wlo, blo, Wro, Wlr, blr, Wrr,
              ind1_o, occ1_o, res1_o):
    ind1_o[...] = jnp.maximum(
        _dot(_mean(fam), Wlm[...]) + blm[...][None, :]
        + _dot(xi[...], Wrm[...]), 0.0)
    occ1_o[...] = jnp.maximum(
        _dot(_mean(occ), Wlo[...]) + blo[...][None, :]
        + _dot(xo[...], Wro[...]), 0.0)
    res1_o[...] = jnp.maximum(
        _dot(_mean(res), Wlr[...]) + blr[...][None, :]
        + _dot(xr[...], Wrr[...]), 0.0)


def _tc2_body(fam, occ, res, ind1, occ1, res1,
              Wli, bli, Wri, Wlo, blo, Wro, Wlr, blr, Wrr,
              Po, Pr, Wa, ba, pop2d, ea0, ea1,
              ind2_o, occ2_o, res2_o):
    ind2_o[...] = (_dot(_mean(fam), Wli[...]) + bli[...][None, :]
                   + _dot(ind1[...], Wri[...]))
    occ2 = (_dot(_mean(occ), Wlo[...]) + blo[...][None, :]
            + _dot(occ1[...], Wro[...]))
    res2 = (_dot(_mean(res), Wlr[...]) + blr[...][None, :]
            + _dot(res1[...], Wrr[...]))

    pop = pop2d[...]
    aid = lax.broadcasted_iota(i32, (N_ATT_P, 128), 0)
    eio = lax.broadcasted_iota(i32, (N_ATT_P, 128), 1)
    lp = jnp.full((N_ATT_P, 128), -1, i32)
    for k in range(32):
        lp = jnp.maximum(lp, jnp.where(pop[k:k + 1, :] == aid,
                                       eio + k * 128, -1))
    lastpos = jnp.max(lp, axis=1, keepdims=True)
    present = lastpos >= 0

    e0 = jnp.zeros((N_ATT_P, D), f32)
    e1 = jnp.zeros((N_ATT_P, D), f32)
    for k in range(32):
        oh = ((eio + k * 128) == lastpos).astype(f32)
        e0 = e0 + _dot(oh, ea0[pl.ds(k * 128, 128)])
        e1 = e1 + _dot(oh, ea1[pl.ds(k * 128, 128)])

    W1 = Wa[pl.ds(0, 128)]
    W2 = Wa[pl.ds(128, 128)]
    occ_new = (_dot(occ2, _dot(Po[...], W1)) + _dot(e0, _dot(Po[...], W2))
               + ba[...][None, :])
    res_new = (_dot(res2, _dot(Pr[...], W1)) + _dot(e1, _dot(Pr[...], W2))
               + ba[...][None, :])
    occ2_o[...] = jnp.where(present, occ_new, occ2)
    res2_o[...] = jnp.where(present, res_new, res2)


def _prep_edges(src, dst, nchunk, dummy):
    tot = 16 * nchunk * EC
    e = src.shape[0]
    src_p = jnp.concatenate([src.astype(i32), jnp.zeros((tot - e,), i32)])
    dst_p = jnp.concatenate([dst.astype(i32), jnp.full((tot - e,), dummy, i32)])
    return src_p.reshape(16, nchunk, EC), dst_p.reshape(16, nchunk, EC)


def _run_round(table, fs, fd, osrc, odst, rsrc, rdst, zrows, ones8):
    return _sc_agg(table, fs, fd, osrc, odst, rsrc, rdst, zrows, ones8)


def kernel(x_individuals, x_occupation, x_residence, edge_index_family,
           edge_index_occupation, edge_index_residence, population,
           edge_attributes, Wl_dir_occ, bl_dir_occ, Wr_dir_occ, Wl_dir_res,
           bl_dir_res, Wr_dir_res, Wl_msg, bl_msg, Wr_msg, Wl_inv_occ,
           bl_inv_occ, Wr_inv_occ, Wl_inv_res, bl_inv_res, Wr_inv_res,
           Wl_inv_ind, bl_inv_ind, Wr_inv_ind, P_occ, P_res, W_aggr, b_aggr):
    n_ind = x_individuals.shape[0]
    n_att = x_occupation.shape[0]

    xi = jnp.pad(x_individuals, ((0, N_IND_P - n_ind), (0, 0)))
    xo = jnp.pad(x_occupation, ((0, N_ATT_P - n_att), (0, 0)))
    xr = jnp.pad(x_residence, ((0, N_ATT_P - n_att), (0, 0)))
    fs1, fd1 = _prep_edges(edge_index_family[1], edge_index_family[0],
                           FAM_CH, n_ind)
    fs2, fd2 = _prep_edges(edge_index_family[0], edge_index_family[1],
                           FAM_CH, n_ind)
    osrc, odst = _prep_edges(edge_index_occupation[0],
                             edge_index_occupation[1], ATT_CH, n_att)
    rsrc, rdst = _prep_edges(edge_index_residence[0],
                             edge_index_residence[1], ATT_CH, n_att)
    zrows = jnp.zeros((640, D), f32)
    ones8 = jnp.ones((EC, D), f32)
    pop2d = population.astype(i32).reshape(32, 128)
    ea0 = edge_attributes[:, 0, :]
    ea1 = edge_attributes[:, 1, :]

    fam, occ, res = _run_round(
        xi, fs1, fd1, osrc, odst, rsrc, rdst, zrows, ones8)
    ind1, occ1, res1 = pl.pallas_call(
        _tc1_body,
        out_shape=(jax.ShapeDtypeStruct((N_IND_P, D), f32),
                   jax.ShapeDtypeStruct((N_ATT_P, D), f32),
                   jax.ShapeDtypeStruct((N_ATT_P, D), f32)),
    )(fam, occ, res, xi, xo, xr,
      Wl_msg, bl_msg, Wr_msg, Wl_dir_occ, bl_dir_occ, Wr_dir_occ,
      Wl_dir_res, bl_dir_res, Wr_dir_res)

    fam2, occ2s, res2s = _run_round(
        ind1, fs2, fd2, osrc, odst, rsrc, rdst, zrows, ones8)
    ind2, occ2, res2 = pl.pallas_call(
        _tc2_body,
        out_shape=(jax.ShapeDtypeStruct((N_IND_P, D), f32),
                   jax.ShapeDtypeStruct((N_ATT_P, D), f32),
                   jax.ShapeDtypeStruct((N_ATT_P, D), f32)),
    )(fam2, occ2s, res2s, ind1, occ1, res1,
      Wl_inv_ind, bl_inv_ind, Wr_inv_ind, Wl_inv_occ, bl_inv_occ, Wr_inv_occ,
      Wl_inv_res, bl_inv_res, Wr_inv_res,
      P_occ, P_res, W_aggr, b_aggr, pop2d, ea0, ea1)

    return (ind2[:n_ind], occ2[:n_att], res2[:n_att])


# speedup vs baseline: 1.7839x; 1.1285x over previous
import jax
import jax.numpy as jnp
from jax import lax
from jax.experimental import pallas as pl
from jax.experimental.pallas import tpu as pltpu
from jax.experimental.pallas import tpu_sc as plsc

f32 = jnp.float32
i32 = jnp.int32

D = 128
N_IND_P = 10240
N_ATT_P = 1024
EC = 64
FAM_CH = 320
ATT_CH = 12


def _sc_agg_body(table, fsrc, fdst, osrc, odst, rsrc, rdst, zrows, ones8,
                 out_fam, out_occ, out_res,
                 acc_fam, acc_occ, acc_res,
                 isrc_g, idst_g, icur_s, icur_d, vbuf, ones_v, sem):
    c = lax.axis_index("c")
    s = lax.axis_index("s")

    pltpu.sync_copy(zrows, acc_fam.at[pl.ds(s * 640, 640)])
    pltpu.sync_copy(zrows.at[pl.ds(0, 64)], acc_occ.at[pl.ds(s * 64, 64)])
    pltpu.sync_copy(zrows.at[pl.ds(0, 64)], acc_res.at[pl.ds(s * 64, 64)])
    pltpu.sync_copy(ones8, ones_v)
    plsc.subcore_barrier()

    def run_edges(src_hbm, dst_hbm, acc, nchunk, gsz):
        def outer(g, carry):
            pltpu.sync_copy(src_hbm.at[s, pl.ds(g * gsz, gsz)],
                            isrc_g.at[pl.ds(0, gsz)])
            pltpu.sync_copy(dst_hbm.at[s, pl.ds(g * gsz, gsz)],
                            idst_g.at[pl.ds(0, gsz)])

            def inner(jj, carry2):
                for l in range(EC // 16):
                    icur_s[pl.ds(l * 16, 16)] = isrc_g[jj, pl.ds(l * 16, 16)]
                    icur_d[pl.ds(l * 16, 16)] = idst_g[jj, pl.ds(l * 16, 16)]

                @pl.when(c == 0)
                def _():
                    pltpu.async_copy(table.at[icur_s], vbuf, sem).wait()
                    pltpu.sync_copy(vbuf, acc.at[icur_d], add=True)

                @pl.when(c == 1)
                def _():
                    pltpu.sync_copy(ones_v, acc.at[icur_d], add=True)

                return carry2

            lax.fori_loop(0, gsz, inner, 0)
            return carry

        lax.fori_loop(0, nchunk // gsz, outer, 0)

    run_edges(fsrc, fdst, acc_fam, FAM_CH, 4)
    run_edges(osrc, odst, acc_occ, ATT_CH, 4)
    run_edges(rsrc, rdst, acc_res, ATT_CH, 4)
    plsc.subcore_barrier()

    pltpu.sync_copy(acc_fam.at[pl.ds(s * 640, 640)],
                    out_fam.at[c, pl.ds(s * 640, 640)])
    pltpu.sync_copy(acc_occ.at[pl.ds(s * 64, 64)],
                    out_occ.at[c, pl.ds(s * 64, 64)])
    pltpu.sync_copy(acc_res.at[pl.ds(s * 64, 64)],
                    out_res.at[c, pl.ds(s * 64, 64)])


_sc_agg = pl.kernel(
    _sc_agg_body,
    out_type=(
        jax.ShapeDtypeStruct((2, N_IND_P, D), f32),
        jax.ShapeDtypeStruct((2, N_ATT_P, D), f32),
        jax.ShapeDtypeStruct((2, N_ATT_P, D), f32),
    ),
    mesh=plsc.VectorSubcoreMesh(core_axis_name="c", subcore_axis_name="s",
                                num_cores=2, num_subcores=16),
    scratch_types=[
        pltpu.VMEM_SHARED((N_IND_P, D), f32),
        pltpu.VMEM_SHARED((N_ATT_P, D), f32),
        pltpu.VMEM_SHARED((N_ATT_P, D), f32),
        pltpu.VMEM((4, EC), i32),
        pltpu.VMEM((4, EC), i32),
        pltpu.VMEM((EC,), i32),
        pltpu.VMEM((EC,), i32),
        pltpu.VMEM((EC, D), f32),
        pltpu.VMEM((EC, D), f32),
        pltpu.SemaphoreType.DMA,
    ],
)


def _mean(ref):
    return ref[0] * (1.0 / jnp.maximum(ref[1][:, 0:1], 1.0))


def _dot(a, b):
    return jax.lax.dot(a, b, preferred_element_type=f32)


def _tc1_body(fam, occ, res, xi, xo, xr,
              Wlm, blm, Wrm, Wlo, blo, Wro, Wlr, blr, Wrr,
              ind1_o, occ1_o, res1_o):
    ind1_o[...] = jnp.maximum(
        _dot(_mean(fam), Wlm[...]) + blm[...][None, :]
        + _dot(xi[...], Wrm[...]), 0.0)
    occ1_o[...] = jnp.maximum(
        _dot(_mean(occ), Wlo[...]) + blo[...][None, :]
        + _dot(xo[...], Wro[...]), 0.0)
    res1_o[...] = jnp.maximum(
        _dot(_mean(res), Wlr[...]) + blr[...][None, :]
        + _dot(xr[...], Wrr[...]), 0.0)


def _tc2_body(fam, occ, res, ind1, occ1, res1,
              Wli, bli, Wri, Wlo, blo, Wro, Wlr, blr, Wrr,
              Po, Pr, Wa, ba, pop2d, ea0, ea1,
              ind2_o, occ2_o, res2_o):
    ind2_o[...] = (_dot(_mean(fam), Wli[...]) + bli[...][None, :]
                   + _dot(ind1[...], Wri[...]))
    occ2 = (_dot(_mean(occ), Wlo[...]) + blo[...][None, :]
            + _dot(occ1[...], Wro[...]))
    res2 = (_dot(_mean(res), Wlr[...]) + blr[...][None, :]
            + _dot(res1[...], Wrr[...]))

    pop = pop2d[...]
    aid = lax.broadcasted_iota(i32, (N_ATT_P, 128), 0)
    eio = lax.broadcasted_iota(i32, (N_ATT_P, 128), 1)
    lp = jnp.full((N_ATT_P, 128), -1, i32)
    for k in range(32):
        lp = jnp.maximum(lp, jnp.where(pop[k:k + 1, :] == aid,
                                       eio + k * 128, -1))
    lastpos = jnp.max(lp, axis=1, keepdims=True)
    present = lastpos >= 0

    e0 = jnp.zeros((N_ATT_P, D), f32)
    e1 = jnp.zeros((N_ATT_P, D), f32)
    for k in range(32):
        oh = ((eio + k * 128) == lastpos).astype(f32)
        e0 = e0 + _dot(oh, ea0[pl.ds(k * 128, 128)])
        e1 = e1 + _dot(oh, ea1[pl.ds(k * 128, 128)])

    W1 = Wa[pl.ds(0, 128)]
    W2 = Wa[pl.ds(128, 128)]
    occ_new = (_dot(occ2, _dot(Po[...], W1)) + _dot(e0, _dot(Po[...], W2))
               + ba[...][None, :])
    res_new = (_dot(res2, _dot(Pr[...], W1)) + _dot(e1, _dot(Pr[...], W2))
               + ba[...][None, :])
    occ2_o[...] = jnp.where(present, occ_new, occ2)
    res2_o[...] = jnp.where(present, res_new, res2)


def _prep_edges(src, dst, nchunk, dummy):
    tot = 16 * nchunk * EC
    e = src.shape[0]
    src_p = jnp.concatenate([src.astype(i32), jnp.zeros((tot - e,), i32)])
    dst_p = jnp.concatenate([dst.astype(i32), jnp.full((tot - e,), dummy, i32)])
    return src_p.reshape(16, nchunk, EC), dst_p.reshape(16, nchunk, EC)


def _run_round(table, fs, fd, osrc, odst, rsrc, rdst, zrows, ones8):
    return _sc_agg(table, fs, fd, osrc, odst, rsrc, rdst, zrows, ones8)


def kernel(x_individuals, x_occupation, x_residence, edge_index_family,
           edge_index_occupation, edge_index_residence, population,
           edge_attributes, Wl_dir_occ, bl_dir_occ, Wr_dir_occ, Wl_dir_res,
           bl_dir_res, Wr_dir_res, Wl_msg, bl_msg, Wr_msg, Wl_inv_occ,
           bl_inv_occ, Wr_inv_occ, Wl_inv_res, bl_inv_res, Wr_inv_res,
           Wl_inv_ind, bl_inv_ind, Wr_inv_ind, P_occ, P_res, W_aggr, b_aggr):
    n_ind = x_individuals.shape[0]
    n_att = x_occupation.shape[0]

    xi = jnp.pad(x_individuals, ((0, N_IND_P - n_ind), (0, 0)))
    xo = jnp.pad(x_occupation, ((0, N_ATT_P - n_att), (0, 0)))
    xr = jnp.pad(x_residence, ((0, N_ATT_P - n_att), (0, 0)))
    fs1, fd1 = _prep_edges(edge_index_family[1], edge_index_family[0],
                           FAM_CH, n_ind)
    fs2, fd2 = _prep_edges(edge_index_family[0], edge_index_family[1],
                           FAM_CH, n_ind)
    osrc, odst = _prep_edges(edge_index_occupation[0],
                             edge_index_occupation[1], ATT_CH, n_att)
    rsrc, rdst = _prep_edges(edge_index_residence[0],
                             edge_index_residence[1], ATT_CH, n_att)
    zrows = jnp.zeros((640, D), f32)
    ones8 = jnp.ones((EC, D), f32)
    pop2d = population.astype(i32).reshape(32, 128)
    ea0 = edge_attributes[:, 0, :]
    ea1 = edge_attributes[:, 1, :]

    fam, occ, res = _run_round(
        xi, fs1, fd1, osrc, odst, rsrc, rdst, zrows, ones8)
    ind1, occ1, res1 = pl.pallas_call(
        _tc1_body,
        out_shape=(jax.ShapeDtypeStruct((N_IND_P, D), f32),
                   jax.ShapeDtypeStruct((N_ATT_P, D), f32),
                   jax.ShapeDtypeStruct((N_ATT_P, D), f32)),
    )(fam, occ, res, xi, xo, xr,
      Wl_msg, bl_msg, Wr_msg, Wl_dir_occ, bl_dir_occ, Wr_dir_occ,
      Wl_dir_res, bl_dir_res, Wr_dir_res)

    fam2, occ2s, res2s = _run_round(
        ind1, fs2, fd2, osrc, odst, rsrc, rdst, zrows, ones8)
    ind2, occ2, res2 = pl.pallas_call(
        _tc2_body,
        out_shape=(jax.ShapeDtypeStruct((N_IND_P, D), f32),
                   jax.ShapeDtypeStruct((N_ATT_P, D), f32),
                   jax.ShapeDtypeStruct((N_ATT_P, D), f32)),
    )(fam2, occ2s, res2s, ind1, occ1, res1,
      Wl_inv_ind, bl_inv_ind, Wr_inv_ind, Wl_inv_occ, bl_inv_occ, Wr_inv_occ,
      Wl_inv_res, bl_inv_res, Wr_inv_res,
      P_occ, P_res, W_aggr, b_aggr, pop2d, ea0, ea1)

    return (ind2[:n_ind], occ2[:n_att], res2[:n_att])
